# speedup vs baseline: 1.6663x; 1.6663x over previous
import jax
import jax.numpy as jnp
from jax.experimental import pallas as pl

B, S, D = 4, 8192, 1024
BLK_S = 1024


def _add_kernel(x_ref, pe_ref, o_ref):
    o_ref[...] = x_ref[...] + pe_ref[...]


def kernel(x, pe):
    grid = (S // BLK_S, B)
    return pl.pallas_call(
        _add_kernel,
        grid=grid,
        in_specs=[
            pl.BlockSpec((1, BLK_S, D), lambda i, j: (j, i, 0)),
            pl.BlockSpec((BLK_S, D), lambda i, j: (i, 0)),
        ],
        out_specs=pl.BlockSpec((1, BLK_S, D), lambda i, j: (j, i, 0)),
        out_shape=jax.ShapeDtypeStruct((B, S, D), x.dtype),
    )(x, pe)


# speedup vs baseline: 1.7379x; 1.0430x over previous
import jax
import jax.numpy as jnp
from jax.experimental import pallas as pl

B, S, D = 4, 8192, 1024
BLK_S = 2048


def _add_kernel(x_ref, pe_ref, o_ref):
    o_ref[...] = x_ref[...] + pe_ref[...]


def kernel(x, pe):
    grid = (S // BLK_S, B)
    return pl.pallas_call(
        _add_kernel,
        grid=grid,
        in_specs=[
            pl.BlockSpec((1, BLK_S, D), lambda i, j: (j, i, 0)),
            pl.BlockSpec((BLK_S, D), lambda i, j: (i, 0)),
        ],
        out_specs=pl.BlockSpec((1, BLK_S, D), lambda i, j: (j, i, 0)),
        out_shape=jax.ShapeDtypeStruct((B, S, D), x.dtype),
    )(x, pe)
